# Initial kernel scaffold; baseline (speedup 1.0000x reference)
import jax
import jax.numpy as jnp
from jax.experimental import pallas as pl


def kernel(node_index, edge_types, graph_table, edge_table):
    raise NotImplementedError("write your pallas kernel here")


# speedup vs baseline: 14.1350x; 14.1350x over previous
import functools

import jax
import jax.numpy as jnp
from jax import lax
from jax.experimental import pallas as pl
from jax.experimental.pallas import tpu as pltpu
from jax.experimental.pallas import tpu_sc as plsc

_NC = 2
_NS = 16
_LANE = 128


def _graph_gather(table, idx):
    n_idx = idx.shape[0]
    d = table.shape[1]
    nw = _NC * _NS
    per_w = n_idx // nw
    mesh = plsc.VectorSubcoreMesh(core_axis_name="c", subcore_axis_name="s")

    @functools.partial(
        pl.kernel,
        mesh=mesh,
        out_type=jax.ShapeDtypeStruct((n_idx, d), jnp.float32),
        scratch_types=[
            pltpu.VMEM((per_w,), jnp.int32),
            pltpu.VMEM((per_w, d), jnp.float32),
            pltpu.SemaphoreType.DMA,
        ],
        compiler_params=pltpu.CompilerParams(use_tc_tiling_on_sc=False),
    )
    def k(table_hbm, idx_hbm, out_hbm, idx_v, rows_v, sem):
        wid = lax.axis_index("s") * _NC + lax.axis_index("c")
        base = wid * per_w
        pltpu.sync_copy(idx_hbm.at[pl.ds(base, per_w)], idx_v)
        pltpu.async_copy(table_hbm.at[idx_v], rows_v, sem).wait()
        pltpu.sync_copy(rows_v, out_hbm.at[pl.ds(base, per_w)])

    return k(table, idx)


def _tc_body(e_ref, g_ref, gt_ref, et_ref, out_ref):
    ti = e_ref.shape[1]
    h_dim = gt_ref.shape[1]
    e = e_ref[0]
    g = g_ref[0]
    gt = gt_ref[0]
    for h in range(h_dim):
        tab = jnp.broadcast_to(et_ref[h : h + 1, :], (ti, _LANE))
        lut = jnp.take_along_axis(tab, e, axis=1)
        gi = g[:, h : h + 1]
        gj = gt[h : h + 1, :]
        out_ref[0, h] = gi * gj + lut


def kernel(node_index, edge_types, graph_table, edge_table):
    b, n = node_index.shape
    h_dim = graph_table.shape[1]

    idx = node_index.reshape(-1).astype(jnp.int32)
    g = _graph_gather(graph_table, idx)
    g3 = g.reshape(b, n, h_dim)
    gt = jnp.transpose(g3, (0, 2, 1))

    et = jnp.zeros((h_dim, _LANE), jnp.float32).at[:, : edge_table.shape[0]].set(
        edge_table.astype(jnp.float32).T
    )

    ti = 128
    grid = (b, n // ti)
    out = pl.pallas_call(
        _tc_body,
        grid=grid,
        in_specs=[
            pl.BlockSpec((1, ti, n), lambda bb, ii: (bb, ii, 0)),
            pl.BlockSpec((1, ti, h_dim), lambda bb, ii: (bb, ii, 0)),
            pl.BlockSpec((1, h_dim, n), lambda bb, ii: (bb, 0, 0)),
            pl.BlockSpec((h_dim, _LANE), lambda bb, ii: (0, 0)),
        ],
        out_specs=pl.BlockSpec((1, h_dim, ti, n), lambda bb, ii: (bb, 0, ii, 0)),
        out_shape=jax.ShapeDtypeStruct((b, h_dim, n, n), jnp.float32),
    )(edge_types, g3, gt, et)
    return out
